# baseline (speedup 1.0000x reference)
import functools

import jax
import jax.numpy as jnp
from jax import lax
from jax.experimental import pallas as pl
from jax.experimental.pallas import tpu as pltpu
from jax.experimental.pallas import tpu_sc as plsc

_CHUNK = 32
_NBUF = 3


@functools.lru_cache(maxsize=None)
def _make_sc_broadcast(B, S, D, dtype):
    info = plsc.get_sparse_core_info()
    nc, ns = info.num_cores, info.num_subcores
    nw = nc * ns
    assert S % (nw * _CHUNK) == 0
    rows_per_w = S // nw
    nchunks = rows_per_w // _CHUNK
    mesh = plsc.VectorSubcoreMesh(core_axis_name="c", subcore_axis_name="s")

    @functools.partial(
        pl.kernel,
        out_type=jax.ShapeDtypeStruct((B, S, D), dtype),
        mesh=mesh,
        scratch_types=[
            pltpu.VMEM_SHARED((ns, _NBUF, _CHUNK, D), dtype),
            pltpu.SemaphoreType.DMA,
            pltpu.SemaphoreType.DMA,
        ],
    )
    def k(table_hbm, out_hbm, buf, rsem, wsem):
        sid = lax.axis_index("s")
        wid = sid * nc + lax.axis_index("c")
        base = wid * rows_per_w

        def read(i):
            return pltpu.async_copy(
                table_hbm.at[pl.ds(base + i * _CHUNK, _CHUNK)],
                buf.at[sid, i % _NBUF],
                rsem,
            )

        def write(i):
            return [
                pltpu.async_copy(
                    buf.at[sid, i % _NBUF],
                    out_hbm.at[b, pl.ds(base + i * _CHUNK, _CHUNK)],
                    wsem,
                )
                for b in range(B)
            ]

        writes = {}
        rd = read(0)
        for i in range(nchunks):
            if i + 1 < nchunks:
                if i + 1 - _NBUF >= 0:
                    for c in writes.pop(i + 1 - _NBUF):
                        c.wait()
                nxt = read(i + 1)
            rd.wait()
            writes[i] = write(i)
            if i + 1 < nchunks:
                rd = nxt
        for ws in writes.values():
            for c in ws:
                c.wait()

    return k


def kernel(x, pos_emb):
    B, S = x.shape
    M, D = pos_emb.shape
    assert S <= M
    return _make_sc_broadcast(B, S, D, pos_emb.dtype)(pos_emb)
